# baseline (speedup 1.0000x reference)
import jax
import jax.numpy as jnp
import numpy as np
from jax.experimental import pallas as pl
from jax.experimental.pallas import tpu as pltpu

_N_OUT = 50
_TEMPLATE = 10
_N_PAIRS = (_N_OUT * (_N_OUT - 1)) // 2
_N_PAIRS_PAD = 1232

_IU, _JU = np.triu_indices(_N_OUT, k=1)
_OHI = np.zeros((_N_PAIRS_PAD, _N_OUT), dtype=np.float32)
_OHJ = np.zeros((_N_PAIRS_PAD, _N_OUT), dtype=np.float32)
_OHI[np.arange(_N_PAIRS), _IU] = 1.0
_OHJ[np.arange(_N_PAIRS), _JU] = 1.0


def _body(idx_ref, cf_ref, noise_ref, W1_ref, b1_ref, W2_ref, b2_ref,
          W3_ref, b3_ref, We1_ref, be1_ref, We2r_ref, be2_ref,
          ohi_ref, ohj_ref, tf_out_ref, ep_out_ref, acc_ref):
    i = pl.program_id(0)

    @pl.when(i == 0)
    def _():
        acc_ref[...] = cf_ref[0]

    @pl.when(jnp.logical_and(i > 0, i < _TEMPLATE))
    def _():
        acc_ref[...] += cf_ref[0]

    @pl.when(i == _TEMPLATE)
    def _():
        p = acc_ref[...] * (1.0 / _TEMPLATE)
        h1 = jnp.maximum(
            jnp.dot(p, W1_ref[...], preferred_element_type=jnp.float32)
            + b1_ref[...], 0.0)
        h2 = jnp.maximum(
            jnp.dot(h1, W2_ref[...], preferred_element_type=jnp.float32)
            + b2_ref[...], 0.0)
        h3 = jax.nn.sigmoid(
            jnp.dot(h2, W3_ref[...], preferred_element_type=jnp.float32)
            + b3_ref[...])

        base = jnp.broadcast_to(h3, (_N_OUT, 512))
        noise_full = jnp.concatenate(
            [jnp.zeros((_TEMPLATE, 512), jnp.float32), noise_ref[...]],
            axis=0)
        tf = base + 0.1 * noise_full
        tf_out_ref[...] = tf

        A = jnp.dot(tf, We1_ref[0:512, :],
                    preferred_element_type=jnp.float32)
        B = jnp.dot(tf, We1_ref[512:1024, :],
                    preferred_element_type=jnp.float32)
        Ai = jnp.dot(ohi_ref[...], A, preferred_element_type=jnp.float32)
        Bj = jnp.dot(ohj_ref[...], B, preferred_element_type=jnp.float32)
        e = jnp.maximum(Ai + Bj + be1_ref[...], 0.0)
        s = jnp.sum(e * We2r_ref[...], axis=1, keepdims=True) + be2_ref[...]
        ep_out_ref[...] = jax.nn.sigmoid(s)


def kernel(clean_features, selected_nodes, noise, W1, b1, W2, b2, W3, b3,
           We1, be1, We2, be2):
    const = lambda i, idx_ref: (0, 0)
    grid_spec = pltpu.PrefetchScalarGridSpec(
        num_scalar_prefetch=1,
        grid=(_TEMPLATE + 1,),
        in_specs=[
            pl.BlockSpec((1, 1, 512),
                         lambda i, idx_ref: (idx_ref[jnp.minimum(i, _TEMPLATE - 1)], 0, 0)),
            pl.BlockSpec((_N_OUT - _TEMPLATE, 512), const),
            pl.BlockSpec((512, 64), const),
            pl.BlockSpec((1, 64), const),
            pl.BlockSpec((64, 64), const),
            pl.BlockSpec((1, 64), const),
            pl.BlockSpec((64, 512), const),
            pl.BlockSpec((1, 512), const),
            pl.BlockSpec((1024, 64), const),
            pl.BlockSpec((1, 64), const),
            pl.BlockSpec((1, 64), const),
            pl.BlockSpec((1, 1), const),
            pl.BlockSpec((_N_PAIRS_PAD, _N_OUT), const),
            pl.BlockSpec((_N_PAIRS_PAD, _N_OUT), const),
        ],
        out_specs=[
            pl.BlockSpec((_N_OUT, 512), const),
            pl.BlockSpec((_N_PAIRS_PAD, 1), const),
        ],
        scratch_shapes=[pltpu.VMEM((1, 512), jnp.float32)],
    )
    tf, ep = pl.pallas_call(
        _body,
        grid_spec=grid_spec,
        out_shape=[
            jax.ShapeDtypeStruct((_N_OUT, 512), jnp.float32),
            jax.ShapeDtypeStruct((_N_PAIRS_PAD, 1), jnp.float32),
        ],
    )(selected_nodes, clean_features.reshape(100000, 1, 512), noise,
      W1, b1.reshape(1, 64), W2, b2.reshape(1, 64), W3, b3.reshape(1, 512),
      We1, be1.reshape(1, 64), We2.reshape(1, 64), be2.reshape(1, 1),
      jnp.asarray(_OHI), jnp.asarray(_OHJ))
    return (tf, ep[:_N_PAIRS])


# speedup vs baseline: 13.9937x; 13.9937x over previous
import jax
import jax.numpy as jnp
import numpy as np
from jax.experimental import pallas as pl
from jax.experimental.pallas import tpu as pltpu

_N_OUT = 50
_TEMPLATE = 10
_N_PAIRS = (_N_OUT * (_N_OUT - 1)) // 2
_N_PAIRS_PAD = 1232

_IU, _JU = np.triu_indices(_N_OUT, k=1)
_OHI = np.zeros((_N_PAIRS_PAD, _N_OUT), dtype=np.float32)
_OHJ = np.zeros((_N_PAIRS_PAD, _N_OUT), dtype=np.float32)
_OHI[np.arange(_N_PAIRS), _IU] = 1.0
_OHJ[np.arange(_N_PAIRS), _JU] = 1.0


def _body(cf_hbm, sel_ref, noise_ref, W1_ref, b1_ref, W2_ref, b2_ref,
          W3_ref, b3_ref, We1_ref, be1_ref, We2r_ref, be2_ref,
          ohi_ref, ohj_ref, tf_out_ref, ep_out_ref, rows_ref, sem):
    copies = [
        pltpu.make_async_copy(
            cf_hbm.at[pl.ds(sel_ref[i], 1), :],
            rows_ref.at[pl.ds(i, 1), :],
            sem)
        for i in range(_TEMPLATE)
    ]
    for c in copies:
        c.start()
    for c in copies:
        c.wait()

    p = jnp.sum(rows_ref[...], axis=0, keepdims=True) * (1.0 / _TEMPLATE)
    h1 = jnp.maximum(
        jnp.dot(p, W1_ref[...], preferred_element_type=jnp.float32)
        + b1_ref[...], 0.0)
    h2 = jnp.maximum(
        jnp.dot(h1, W2_ref[...], preferred_element_type=jnp.float32)
        + b2_ref[...], 0.0)
    h3 = jax.nn.sigmoid(
        jnp.dot(h2, W3_ref[...], preferred_element_type=jnp.float32)
        + b3_ref[...])

    base = jnp.broadcast_to(h3, (_N_OUT, 512))
    noise_full = jnp.concatenate(
        [jnp.zeros((_TEMPLATE, 512), jnp.float32), noise_ref[...]],
        axis=0)
    tf = base + 0.1 * noise_full
    tf_out_ref[...] = tf

    A = jnp.dot(tf, We1_ref[0:512, :],
                preferred_element_type=jnp.float32)
    B = jnp.dot(tf, We1_ref[512:1024, :],
                preferred_element_type=jnp.float32)
    Ai = jnp.dot(ohi_ref[...], A, preferred_element_type=jnp.float32)
    Bj = jnp.dot(ohj_ref[...], B, preferred_element_type=jnp.float32)
    e = jnp.maximum(Ai + Bj + be1_ref[...], 0.0)
    s = jnp.sum(e * We2r_ref[...], axis=1, keepdims=True) + be2_ref[...]
    ep_out_ref[...] = jax.nn.sigmoid(s)


def kernel(clean_features, selected_nodes, noise, W1, b1, W2, b2, W3, b3,
           We1, be1, We2, be2):
    vmem = pl.BlockSpec(memory_space=pltpu.VMEM)
    tf, ep = pl.pallas_call(
        _body,
        in_specs=[
            pl.BlockSpec(memory_space=pl.ANY),
            pl.BlockSpec(memory_space=pltpu.SMEM),
            vmem, vmem, vmem, vmem, vmem, vmem, vmem,
            vmem, vmem, vmem, vmem, vmem, vmem,
        ],
        out_specs=[vmem, vmem],
        out_shape=[
            jax.ShapeDtypeStruct((_N_OUT, 512), jnp.float32),
            jax.ShapeDtypeStruct((_N_PAIRS_PAD, 1), jnp.float32),
        ],
        scratch_shapes=[
            pltpu.VMEM((_TEMPLATE, 512), jnp.float32),
            pltpu.SemaphoreType.DMA,
        ],
    )(clean_features, selected_nodes[:_TEMPLATE], noise,
      W1, b1.reshape(1, 64), W2, b2.reshape(1, 64), W3, b3.reshape(1, 512),
      We1, be1.reshape(1, 64), We2.reshape(1, 64), be2.reshape(1, 1),
      jnp.asarray(_OHI), jnp.asarray(_OHJ))
    return (tf, ep[:_N_PAIRS])
